# baseline (speedup 1.0000x reference)
import functools

import jax
import jax.numpy as jnp
from jax import lax
from jax.experimental import pallas as pl
from jax.experimental.pallas import tpu as pltpu
from jax.experimental.pallas import tpu_sc as plsc

B = 16384
WORD = 26
NF = 13
ROWS = WORD + NF
D = 64
NC, NS, L = 2, 16, 16
NW = NC * NS
G = 128
CAT_GROUPS = B * WORD // G
NUM_GROUPS = B * NF // G
CG_PER_W = CAT_GROUPS // NW
NG_PER_W = NUM_GROUPS // NW

_mesh = plsc.VectorSubcoreMesh(core_axis_name="c", subcore_axis_name="s")


@functools.partial(
    pl.kernel,
    out_type=jax.ShapeDtypeStruct((B * ROWS, D), jnp.float32),
    mesh=_mesh,
    scratch_types=[
        pltpu.VMEM((CG_PER_W, 1, G), jnp.int32),
        pltpu.VMEM((CG_PER_W, 1, G), jnp.int32),
        pltpu.VMEM((NG_PER_W, 1, G), jnp.int32),
        pltpu.VMEM((NG_PER_W, 1, G), jnp.float32),
        pltpu.VMEM((D,), jnp.float32),
        pltpu.VMEM((D,), jnp.float32),
        pltpu.VMEM((G, D), jnp.float32),
        pltpu.SemaphoreType.DMA,
        pltpu.SemaphoreType.DMA,
    ],
    compiler_params=pltpu.CompilerParams(use_tc_tiling_on_sc=False),
)
def _emb_kernel(idx_hbm, cdst_hbm, ndst_hbm, xnum_hbm, w8_hbm, b8_hbm,
                lut_hbm, out_hbm, idx_v, cdst_v, ndst_v, xnum_v, w_v, b_v,
                buf, gsem, ssem):
    wid = lax.axis_index("s") * NC + lax.axis_index("c")
    cbase = pl.multiple_of(wid * CG_PER_W, 8)
    nbase = pl.multiple_of(wid * NG_PER_W, 8)
    pltpu.sync_copy(idx_hbm.at[pl.ds(cbase, CG_PER_W)], idx_v)
    pltpu.sync_copy(cdst_hbm.at[pl.ds(cbase, CG_PER_W)], cdst_v)
    pltpu.sync_copy(ndst_hbm.at[pl.ds(nbase, NG_PER_W)], ndst_v)
    pltpu.sync_copy(xnum_hbm.at[pl.ds(nbase, NG_PER_W)], xnum_v)
    pltpu.sync_copy(w8_hbm, w_v)
    pltpu.sync_copy(b8_hbm, b_v)

    w = [w_v[pl.ds(c * L, L)] for c in range(D // L)]
    bb = [b_v[pl.ds(c * L, L)] for c in range(D // L)]

    def cat_group(g, carry):
        pltpu.async_copy(lut_hbm.at[idx_v.at[g, 0]], buf, gsem).wait()

        def scale_row(r, c2):
            for c in range(D // L):
                buf[r, pl.ds(c * L, L)] = buf[r, pl.ds(c * L, L)] * 8.0
            return c2

        lax.fori_loop(0, G, scale_row, 0)
        pltpu.async_copy(buf, out_hbm.at[cdst_v.at[g, 0]], ssem).wait()
        return carry

    lax.fori_loop(0, CG_PER_W, cat_group, 0)

    def num_group(g, carry):
        def fill_rows(r16, c2):
            s_vec = xnum_v[g, 0, pl.ds(r16 * L, L)]
            for l in range(L):
                s = s_vec[l]
                for c in range(D // L):
                    buf[r16 * L + l, pl.ds(c * L, L)] = w[c] * s + bb[c]
            return c2

        lax.fori_loop(0, G // L, fill_rows, 0)
        pltpu.async_copy(buf, out_hbm.at[ndst_v.at[g, 0]], ssem).wait()
        return carry

    lax.fori_loop(0, NG_PER_W, num_group, 0)


def kernel(x, lut, W, b):
    idx = x[:, :WORD].astype(jnp.int32).reshape(CAT_GROUPS, 1, G)
    xnum = x[:, WORD:].reshape(NUM_GROUPS, 1, G)
    f = jnp.arange(B * WORD, dtype=jnp.int32)
    cdst = (f + (f // WORD) * NF).reshape(CAT_GROUPS, 1, G)
    g = jnp.arange(B * NF, dtype=jnp.int32)
    ndst = ((g // NF) * ROWS + WORD + g % NF).reshape(NUM_GROUPS, 1, G)
    w8 = W[0] * 8.0
    b8 = b * 8.0
    outf = _emb_kernel(idx, cdst, ndst, xnum, w8, b8, lut)
    return outf.reshape(B, ROWS, D)


# speedup vs baseline: 1.1367x; 1.1367x over previous
import functools

import jax
import jax.numpy as jnp
from jax import lax
from jax.experimental import pallas as pl
from jax.experimental.pallas import tpu as pltpu
from jax.experimental.pallas import tpu_sc as plsc

B = 16384
WORD = 26
NF = 13
ROWS = WORD + NF
D = 64
NC, NS, L = 2, 16, 16
NW = NC * NS
G = 128
CAT_GROUPS = B * WORD // G
NUM_GROUPS = B * NF // G
CG_PER_W = CAT_GROUPS // NW
NG_PER_W = NUM_GROUPS // NW
NBUF = 4
LEAD = 2

_mesh = plsc.VectorSubcoreMesh(core_axis_name="c", subcore_axis_name="s")


@functools.partial(
    pl.kernel,
    out_type=jax.ShapeDtypeStruct((B * ROWS, D), jnp.float32),
    mesh=_mesh,
    scratch_types=[
        pltpu.VMEM((CG_PER_W, 1, G), jnp.int32),
        pltpu.VMEM((CG_PER_W, 1, G), jnp.int32),
        pltpu.VMEM((NG_PER_W, 1, G), jnp.int32),
        pltpu.VMEM((NG_PER_W, 1, G), jnp.float32),
        pltpu.VMEM((D,), jnp.float32),
        pltpu.VMEM((D,), jnp.float32),
        [pltpu.VMEM((G, D), jnp.float32) for _ in range(NBUF)],
        [pltpu.SemaphoreType.DMA for _ in range(NBUF)],
        [pltpu.SemaphoreType.DMA for _ in range(NBUF)],
    ],
    compiler_params=pltpu.CompilerParams(use_tc_tiling_on_sc=False),
)
def _emb_kernel(idx_hbm, cdst_hbm, ndst_hbm, xnum_hbm, w8_hbm, b8_hbm,
                lut_hbm, out_hbm, idx_v, cdst_v, ndst_v, xnum_v, w_v, b_v,
                bufs, gsems, ssems):
    wid = lax.axis_index("s") * NC + lax.axis_index("c")
    cbase = pl.multiple_of(wid * CG_PER_W, 8)
    nbase = pl.multiple_of(wid * NG_PER_W, 8)
    pltpu.sync_copy(idx_hbm.at[pl.ds(cbase, CG_PER_W)], idx_v)
    pltpu.sync_copy(cdst_hbm.at[pl.ds(cbase, CG_PER_W)], cdst_v)
    pltpu.sync_copy(ndst_hbm.at[pl.ds(nbase, NG_PER_W)], ndst_v)
    pltpu.sync_copy(xnum_hbm.at[pl.ds(nbase, NG_PER_W)], xnum_v)
    pltpu.sync_copy(w8_hbm, w_v)
    pltpu.sync_copy(b8_hbm, b_v)

    w = [w_v[pl.ds(c * L, L)] for c in range(D // L)]
    bb = [b_v[pl.ds(c * L, L)] for c in range(D // L)]

    def start_gather(t, slot):
        pltpu.async_copy(lut_hbm.at[idx_v.at[t, 0]], bufs[slot], gsems[slot])

    def wait_gather(slot):
        pltpu.make_async_copy(lut_hbm.at[idx_v.at[0, 0]], bufs[slot],
                              gsems[slot]).wait()

    def start_scatter(t, slot, dst_v):
        pltpu.async_copy(bufs[slot], out_hbm.at[dst_v.at[t, 0]], ssems[slot])

    def wait_scatter(slot):
        pltpu.make_async_copy(bufs[slot], out_hbm.at[cdst_v.at[0, 0]],
                              ssems[slot]).wait()

    def scale(slot):
        buf = bufs[slot]

        def scale_rows(r8, c2):
            r0 = pl.multiple_of(r8 * 8, 8)
            for l in range(8):
                for c in range(D // L):
                    buf[r0 + l, pl.ds(c * L, L)] = (
                        buf[r0 + l, pl.ds(c * L, L)] * 8.0)
            return c2

        lax.fori_loop(0, G // 8, scale_rows, 0)

    for t in range(LEAD):
        start_gather(t, t % NBUF)

    def cat_outer(t4, carry):
        t0 = t4 * NBUF
        for b in range(NBUF):
            t = t0 + b
            gslot = (b + LEAD) % NBUF
            tg = t + LEAD
            if b < NBUF - LEAD:

                @pl.when(t4 >= 1)
                def _():
                    wait_scatter(gslot)

            else:
                wait_scatter(gslot)
            if (NBUF * (CG_PER_W // NBUF - 1) + b + LEAD) < CG_PER_W:
                start_gather(tg, gslot)
            else:

                @pl.when(t4 < CG_PER_W // NBUF - 1)
                def _():
                    start_gather(tg, gslot)

            wait_gather(b)
            scale(b)
            start_scatter(t, b, cdst_v)
        return carry

    lax.fori_loop(0, CG_PER_W // NBUF, cat_outer, 0)
    for b in range(NBUF - LEAD, NBUF):
        wait_scatter(b)

    def num_outer(t4, carry):
        t0 = t4 * NBUF
        for b in range(NBUF):
            t = t0 + b

            @pl.when(t4 >= 1)
            def _():
                wait_scatter(b)

            buf = bufs[b]

            def fill_rows(r16, c2):
                s_vec = xnum_v[t, 0, pl.ds(r16 * L, L)]
                r0 = pl.multiple_of(r16 * L, 8)
                for l in range(L):
                    s = s_vec[l]
                    for c in range(D // L):
                        buf[r0 + l, pl.ds(c * L, L)] = w[c] * s + bb[c]
                return c2

            lax.fori_loop(0, G // L, fill_rows, 0)
            start_scatter(t, b, ndst_v)
        return carry

    lax.fori_loop(0, NG_PER_W // NBUF, num_outer, 0)
    for b in range(NBUF):
        wait_scatter(b)


def kernel(x, lut, W, b):
    idx = x[:, :WORD].astype(jnp.int32).reshape(CAT_GROUPS, 1, G)
    xnum = x[:, WORD:].reshape(NUM_GROUPS, 1, G)
    f = jnp.arange(B * WORD, dtype=jnp.int32)
    cdst = (f + (f // WORD) * NF).reshape(CAT_GROUPS, 1, G)
    g = jnp.arange(B * NF, dtype=jnp.int32)
    ndst = ((g // NF) * ROWS + WORD + g % NF).reshape(NUM_GROUPS, 1, G)
    w8 = W[0] * 8.0
    b8 = b * 8.0
    outf = _emb_kernel(idx, cdst, ndst, xnum, w8, b8, lut)
    return outf.reshape(B, ROWS, D)
